# Initial kernel scaffold; baseline (speedup 1.0000x reference)
import jax
import jax.numpy as jnp
from jax.experimental import pallas as pl


def kernel(features, structural_indices):
    raise NotImplementedError("write your pallas kernel here")


# speedup vs baseline: 4.8125x; 4.8125x over previous
import functools

import jax
import jax.numpy as jnp
from jax import lax
from jax.experimental import pallas as pl
from jax.experimental.pallas import tpu as pltpu
from jax.experimental.pallas import tpu_sc as plsc

N = 320000
D = 128
S = 10000
NC = 2
NS = 16
NW = NC * NS
CHUNK = 256
IDXR = CHUNK // 128
NCHUNKS = N // CHUNK
ZROWS = S // NS

_mesh = plsc.VectorSubcoreMesh(core_axis_name="c", subcore_axis_name="s")


@functools.partial(
    pl.kernel,
    out_type=jax.ShapeDtypeStruct((NC * S, D), jnp.float32),
    mesh=_mesh,
    scratch_types=[
        pltpu.VMEM((CHUNK, D), jnp.float32),
        pltpu.VMEM((IDXR, 128), jnp.int32),
        pltpu.VMEM_SHARED((S, D), jnp.float32),
    ],
)
def _seg_sum_sc(feat_hbm, idx_hbm, out_hbm, feat_buf, idx_buf, acc):
    c = lax.axis_index("c")
    s = lax.axis_index("s")
    wid = s * NC + c

    zero16 = jnp.zeros((16,), jnp.float32)

    def zrow(i, carry):
        for j in range(D // 16):
            feat_buf[i, pl.ds(j * 16, 16)] = zero16
        return carry

    lax.fori_loop(0, CHUNK, zrow, 0)
    off = pl.multiple_of(s * ZROWS - (s % 8), 8)
    pltpu.sync_copy(feat_buf, acc.at[pl.ds(off, CHUNK)])
    pltpu.sync_copy(feat_buf, acc.at[pl.ds(pl.multiple_of(off + CHUNK, 8), CHUNK)])
    pltpu.sync_copy(feat_buf.at[pl.ds(0, 112)],
                    acc.at[pl.ds(pl.multiple_of(off + 2 * CHUNK, 8), 112)])

    @pl.when(s % 8 == 7)
    def _zero_tail():
        pltpu.sync_copy(feat_buf.at[pl.ds(0, 8)],
                        acc.at[pl.ds(pl.multiple_of(off + 624, 8), 8)])

    plsc.subcore_barrier()

    def chunk_body(i, carry):
        r = wid + i * NW
        base = r * CHUNK
        pltpu.sync_copy(feat_hbm.at[pl.ds(base, CHUNK)], feat_buf)
        for j in range(IDXR):
            pltpu.sync_copy(idx_hbm.at[pl.ds(base + j * 128, 128)],
                            idx_buf.at[j])
        for j in range(IDXR):
            pltpu.sync_copy(feat_buf.at[pl.ds(j * 128, 128)],
                            acc.at[idx_buf.at[j]], add=True)
        return carry

    nchunks_w = (NCHUNKS - wid + NW - 1) // NW
    lax.fori_loop(0, nchunks_w, chunk_body, 0)
    plsc.subcore_barrier()

    obase = pl.multiple_of(c * S + off, 8)
    pltpu.sync_copy(acc.at[pl.ds(off, 624)], out_hbm.at[pl.ds(obase, 624)])

    @pl.when(s % 8 == 7)
    def _dump_tail():
        pltpu.sync_copy(acc.at[pl.ds(pl.multiple_of(off + 624, 8), 8)],
                        out_hbm.at[pl.ds(pl.multiple_of(c * S + off + 624, 8), 8)])


def _add_body(a_ref, b_ref, o_ref):
    o_ref[...] = a_ref[...] + b_ref[...]


_BLK = 1000


def _combine_partials(a, b):
    return pl.pallas_call(
        _add_body,
        out_shape=jax.ShapeDtypeStruct((S, D), jnp.float32),
        grid=(S // _BLK,),
        in_specs=[pl.BlockSpec((_BLK, D), lambda i: (i, 0)),
                  pl.BlockSpec((_BLK, D), lambda i: (i, 0))],
        out_specs=pl.BlockSpec((_BLK, D), lambda i: (i, 0)),
    )(a, b)


@jax.jit
def kernel(features, structural_indices):
    partials = _seg_sum_sc(features, structural_indices)
    return _combine_partials(partials[:S], partials[S:])


# speedup vs baseline: 7.3273x; 1.5226x over previous
import functools

import jax
import jax.numpy as jnp
from jax import lax
from jax.experimental import pallas as pl
from jax.experimental.pallas import tpu as pltpu
from jax.experimental.pallas import tpu_sc as plsc

N = 320000
D = 128
S = 10000
NC = 2
NS = 16
NW = NC * NS
CHUNK = 128
IDXR = CHUNK // 128
NCHUNKS = N // CHUNK
CBASE = NCHUNKS // NW
CEXTRA = NCHUNKS - CBASE * NW
IDX_ROWS = -(-((CBASE + 1) * IDXR + 8) // 8) * 8
IDX_HBM_ROWS = NCHUNKS * IDXR + 8

_mesh = plsc.VectorSubcoreMesh(core_axis_name="c", subcore_axis_name="s")


@functools.partial(
    pl.kernel,
    out_type=jax.ShapeDtypeStruct((NC * S, D), jnp.float32),
    mesh=_mesh,
    scratch_types=[
        pltpu.VMEM((2, CHUNK, D), jnp.float32),
        pltpu.VMEM((IDX_ROWS, 128), jnp.int32),
        pltpu.VMEM_SHARED((S, D), jnp.float32),
        pltpu.SemaphoreType.DMA,
        pltpu.SemaphoreType.DMA,
    ],
)
def _seg_sum_sc(feat_hbm, idx_hbm, out_hbm, feat_bufs, idx_all, acc,
                gsem0, gsem1):
    c = lax.axis_index("c")
    s = lax.axis_index("s")
    wid = s * NC + c
    gsems = (gsem0, gsem1)

    zero16 = jnp.zeros((16,), jnp.float32)

    def zrow(i, carry):
        for j in range(D // 16):
            feat_bufs[0, i, pl.ds(j * 16, 16)] = zero16
        return carry

    lax.fori_loop(0, CHUNK, zrow, 0)
    zbuf = feat_bufs.at[0]
    off = pl.multiple_of(s * (S // NS) - (s % 8), 8)
    for z in range(624 // CHUNK):
        pltpu.sync_copy(zbuf, acc.at[pl.ds(pl.multiple_of(off + z * CHUNK, 8),
                                           CHUNK)])
    zrem = 624 % CHUNK
    if zrem:
        pltpu.sync_copy(zbuf.at[pl.ds(0, zrem)],
                        acc.at[pl.ds(pl.multiple_of(off + 624 - zrem, 8), zrem)])

    @pl.when(s % 8 == 7)
    def _zero_tail():
        pltpu.sync_copy(zbuf.at[pl.ds(0, 8)],
                        acc.at[pl.ds(pl.multiple_of(off + 624, 8), 8)])

    plsc.subcore_barrier()

    cnt = CBASE + (wid < CEXTRA).astype(jnp.int32)
    start = wid * CBASE + jnp.minimum(wid, CEXTRA)
    astart = pl.multiple_of((start * IDXR) - (start * IDXR) % 8, 8)
    d = start * IDXR - astart
    pltpu.sync_copy(idx_hbm.at[pl.ds(astart, IDX_ROWS)], idx_all)

    def issue_gather(b, k):
        rbase = pl.multiple_of((start + k) * CHUNK, 8)
        pltpu.async_copy(feat_hbm.at[pl.ds(rbase, CHUNK)],
                         feat_bufs.at[b], gsems[b])

    def wait_gather(b, k):
        rbase = pl.multiple_of((start + k) * CHUNK, 8)
        pltpu.make_async_copy(feat_hbm.at[pl.ds(rbase, CHUNK)],
                              feat_bufs.at[b], gsems[b]).wait()

    for b in range(2):
        issue_gather(b, b)

    def superstep(i, carry):
        for b in range(2):
            k = i * 2 + b

            @pl.when(k < cnt)
            def _do():
                wait_gather(b, k)
                for j in range(IDXR):
                    pltpu.sync_copy(feat_bufs.at[b, pl.ds(j * 128, 128)],
                                    acc.at[idx_all.at[d + k * IDXR + j]],
                                    add=True)

                @pl.when(k + 2 < cnt)
                def _prefetch():
                    issue_gather(b, k + 2)
        return carry

    lax.fori_loop(0, (CBASE + 2) // 2, superstep, 0)
    plsc.subcore_barrier()

    obase = pl.multiple_of(c * S + off, 8)
    pltpu.sync_copy(acc.at[pl.ds(off, 624)], out_hbm.at[pl.ds(obase, 624)])

    @pl.when(s % 8 == 7)
    def _dump_tail():
        pltpu.sync_copy(acc.at[pl.ds(pl.multiple_of(off + 624, 8), 8)],
                        out_hbm.at[pl.ds(pl.multiple_of(c * S + off + 624, 8), 8)])


def _add_body(a_ref, b_ref, o_ref):
    o_ref[...] = a_ref[...] + b_ref[...]


_BLK = 1000


def _combine_partials(a, b):
    return pl.pallas_call(
        _add_body,
        out_shape=jax.ShapeDtypeStruct((S, D), jnp.float32),
        grid=(S // _BLK,),
        in_specs=[pl.BlockSpec((_BLK, D), lambda i: (i, 0)),
                  pl.BlockSpec((_BLK, D), lambda i: (i, 0))],
        out_specs=pl.BlockSpec((_BLK, D), lambda i: (i, 0)),
    )(a, b)


@jax.jit
def kernel(features, structural_indices):
    idx2d = jnp.pad(structural_indices,
                    (0, IDX_HBM_ROWS * 128 - N)).reshape(IDX_HBM_ROWS, 128)
    partials = _seg_sum_sc(features, idx2d)
    return _combine_partials(partials[:S], partials[S:])


# speedup vs baseline: 7.7185x; 1.0534x over previous
import functools

import jax
import jax.numpy as jnp
from jax import lax
from jax.experimental import pallas as pl
from jax.experimental.pallas import tpu as pltpu
from jax.experimental.pallas import tpu_sc as plsc

N = 320000
D = 128
S = 10000
NC = 2
NS = 16
NW = NC * NS
CHUNK = 128
IDXR = CHUNK // 128
NCHUNKS = N // CHUNK
CBASE = NCHUNKS // NW
CEXTRA = NCHUNKS - CBASE * NW
IDX_ROWS = 88
IDX_HBM_ROWS = -(-NCHUNKS * IDXR // 8) * 8

_mesh = plsc.VectorSubcoreMesh(core_axis_name="c", subcore_axis_name="s")


@functools.partial(
    pl.kernel,
    out_type=jax.ShapeDtypeStruct((NC * S, D), jnp.float32),
    mesh=_mesh,
    scratch_types=[
        pltpu.VMEM((2, CHUNK, D), jnp.float32),
        pltpu.VMEM((IDX_ROWS, 128), jnp.int32),
        pltpu.VMEM_SHARED((S, D), jnp.float32),
        pltpu.SemaphoreType.DMA,
        pltpu.SemaphoreType.DMA,
    ],
)
def _seg_sum_sc(feat_hbm, idx_hbm, out_hbm, feat_bufs, idx_all, acc,
                gsem0, gsem1):
    c = lax.axis_index("c")
    s = lax.axis_index("s")
    wid = s * NC + c
    gsems = (gsem0, gsem1)

    zero16 = jnp.zeros((16,), jnp.float32)

    def zrow(i, carry):
        for j in range(D // 16):
            feat_bufs[0, i, pl.ds(j * 16, 16)] = zero16
        return carry

    lax.fori_loop(0, CHUNK, zrow, 0)
    zbuf = feat_bufs.at[0]
    off = pl.multiple_of(s * (S // NS) - (s % 8), 8)
    for z in range(624 // CHUNK):
        pltpu.sync_copy(zbuf, acc.at[pl.ds(pl.multiple_of(off + z * CHUNK, 8),
                                           CHUNK)])
    zrem = 624 % CHUNK
    if zrem:
        pltpu.sync_copy(zbuf.at[pl.ds(0, zrem)],
                        acc.at[pl.ds(pl.multiple_of(off + 624 - zrem, 8), zrem)])

    @pl.when(s % 8 == 7)
    def _zero_tail():
        pltpu.sync_copy(zbuf.at[pl.ds(0, 8)],
                        acc.at[pl.ds(pl.multiple_of(off + 624, 8), 8)])

    plsc.subcore_barrier()

    cnt = CBASE + (wid < CEXTRA).astype(jnp.int32)
    start = wid * CBASE + jnp.minimum(wid, CEXTRA)
    astart = pl.multiple_of((start * IDXR) - (start * IDXR) % 8, 8)
    d = start * IDXR - astart
    pltpu.sync_copy(idx_hbm.at[pl.ds(astart, IDX_ROWS)], idx_all)

    def issue_gather(b, k):
        rbase = pl.multiple_of((start + k) * CHUNK, 8)
        pltpu.async_copy(feat_hbm.at[pl.ds(rbase, CHUNK)],
                         feat_bufs.at[b], gsems[b])

    def wait_gather(b, k):
        rbase = pl.multiple_of((start + k) * CHUNK, 8)
        pltpu.make_async_copy(feat_hbm.at[pl.ds(rbase, CHUNK)],
                              feat_bufs.at[b], gsems[b]).wait()

    for b in range(2):
        issue_gather(b, b)

    def superstep(i, carry):
        for b in range(2):
            k = i * 2 + b

            @pl.when(k < cnt)
            def _do():
                wait_gather(b, k)
                for j in range(IDXR):
                    pltpu.sync_copy(feat_bufs.at[b, pl.ds(j * 128, 128)],
                                    acc.at[idx_all.at[d + k * IDXR + j]],
                                    add=True)

                @pl.when(k + 2 < cnt)
                def _prefetch():
                    issue_gather(b, k + 2)
        return carry

    lax.fori_loop(0, (CBASE + 2) // 2, superstep, 0)
    plsc.subcore_barrier()

    obase = pl.multiple_of(c * S + off, 8)
    pltpu.sync_copy(acc.at[pl.ds(off, 624)], out_hbm.at[pl.ds(obase, 624)])

    @pl.when(s % 8 == 7)
    def _dump_tail():
        pltpu.sync_copy(acc.at[pl.ds(pl.multiple_of(off + 624, 8), 8)],
                        out_hbm.at[pl.ds(pl.multiple_of(c * S + off + 624, 8), 8)])


def _add_body(a_ref, b_ref, o_ref):
    o_ref[...] = a_ref[...] + b_ref[...]


_BLK = 1000


def _combine_partials(partials):
    return pl.pallas_call(
        _add_body,
        out_shape=jax.ShapeDtypeStruct((S, D), jnp.float32),
        grid=(S // _BLK,),
        in_specs=[pl.BlockSpec((_BLK, D), lambda i: (i, 0)),
                  pl.BlockSpec((_BLK, D), lambda i: (i + S // _BLK, 0))],
        out_specs=pl.BlockSpec((_BLK, D), lambda i: (i, 0)),
    )(partials, partials)


@jax.jit
def kernel(features, structural_indices):
    idx2d = jnp.pad(structural_indices,
                    (0, IDX_HBM_ROWS * 128 - N)).reshape(IDX_HBM_ROWS, 128)
    partials = _seg_sum_sc(features, idx2d)
    return _combine_partials(partials)


# speedup vs baseline: 8.1215x; 1.0522x over previous
import functools

import jax
import jax.numpy as jnp
from jax import lax
from jax.experimental import pallas as pl
from jax.experimental.pallas import tpu as pltpu
from jax.experimental.pallas import tpu_sc as plsc

N = 320000
D = 128
S = 10000
NC = 2
NS = 16
NW = NC * NS
CHUNK = 128
NCHUNKS = N // CHUNK
CBASE = NCHUNKS // NW
CEXTRA = NCHUNKS - CBASE * NW
NBUF = 3

_mesh = plsc.VectorSubcoreMesh(core_axis_name="c", subcore_axis_name="s")


@functools.partial(
    pl.kernel,
    out_type=jax.ShapeDtypeStruct((NC * S, D), jnp.float32),
    mesh=_mesh,
    scratch_types=[
        pltpu.VMEM((NBUF, CHUNK, D), jnp.float32),
        pltpu.VMEM((NBUF, 128), jnp.int32),
        pltpu.VMEM_SHARED((S, D), jnp.float32),
        pltpu.SemaphoreType.DMA,
        pltpu.SemaphoreType.DMA,
        pltpu.SemaphoreType.DMA,
    ],
)
def _seg_sum_sc(feat_hbm, idx_hbm, out_hbm, feat_bufs, idx_bufs, acc,
                gsem0, gsem1, gsem2):
    c = lax.axis_index("c")
    s = lax.axis_index("s")
    wid = s * NC + c
    gsems = (gsem0, gsem1, gsem2)

    zero16 = jnp.zeros((16,), jnp.float32)

    def zrow(i, carry):
        for j in range(D // 16):
            feat_bufs[0, i, pl.ds(j * 16, 16)] = zero16
        return carry

    lax.fori_loop(0, CHUNK, zrow, 0)
    zbuf = feat_bufs.at[0]
    off = pl.multiple_of(s * (S // NS) - (s % 8), 8)
    for z in range(624 // CHUNK):
        pltpu.sync_copy(zbuf, acc.at[pl.ds(pl.multiple_of(off + z * CHUNK, 8),
                                           CHUNK)])
    zrem = 624 % CHUNK
    if zrem:
        pltpu.sync_copy(zbuf.at[pl.ds(0, zrem)],
                        acc.at[pl.ds(pl.multiple_of(off + 624 - zrem, 8), zrem)])

    @pl.when(s % 8 == 7)
    def _zero_tail():
        pltpu.sync_copy(zbuf.at[pl.ds(0, 8)],
                        acc.at[pl.ds(pl.multiple_of(off + 624, 8), 8)])

    plsc.subcore_barrier()

    cnt = CBASE + (wid < CEXTRA).astype(jnp.int32)
    start = wid * CBASE + jnp.minimum(wid, CEXTRA)

    def issue_gather(b, k):
        rbase = pl.multiple_of((start + k) * CHUNK, 8)
        pltpu.async_copy(feat_hbm.at[pl.ds(rbase, CHUNK)],
                         feat_bufs.at[b], gsems[b])
        pltpu.async_copy(idx_hbm.at[pl.ds(rbase, CHUNK)],
                         idx_bufs.at[b], gsems[b])

    def wait_gather(b, k):
        rbase = pl.multiple_of((start + k) * CHUNK, 8)
        pltpu.make_async_copy(feat_hbm.at[pl.ds(rbase, CHUNK)],
                              feat_bufs.at[b], gsems[b]).wait()
        pltpu.make_async_copy(idx_hbm.at[pl.ds(rbase, CHUNK)],
                              idx_bufs.at[b], gsems[b]).wait()

    for b in range(NBUF):
        issue_gather(b, b)

    def superstep(i, carry):
        for b in range(NBUF):
            k = i * NBUF + b

            @pl.when(k < cnt)
            def _do():
                wait_gather(b, k)
                pltpu.sync_copy(feat_bufs.at[b], acc.at[idx_bufs.at[b]],
                                add=True)

                @pl.when(k + NBUF < cnt)
                def _prefetch():
                    issue_gather(b, k + NBUF)
        return carry

    lax.fori_loop(0, (CBASE + NBUF) // NBUF, superstep, 0)
    plsc.subcore_barrier()

    obase = pl.multiple_of(c * S + off, 8)
    pltpu.sync_copy(acc.at[pl.ds(off, 624)], out_hbm.at[pl.ds(obase, 624)])

    @pl.when(s % 8 == 7)
    def _dump_tail():
        pltpu.sync_copy(acc.at[pl.ds(pl.multiple_of(off + 624, 8), 8)],
                        out_hbm.at[pl.ds(pl.multiple_of(c * S + off + 624, 8), 8)])


def _add_body(a_ref, b_ref, o_ref):
    o_ref[...] = a_ref[...] + b_ref[...]


_BLK = 1000


def _combine_partials(partials):
    return pl.pallas_call(
        _add_body,
        out_shape=jax.ShapeDtypeStruct((S, D), jnp.float32),
        grid=(S // _BLK,),
        in_specs=[pl.BlockSpec((_BLK, D), lambda i: (i, 0)),
                  pl.BlockSpec((_BLK, D), lambda i: (i + S // _BLK, 0))],
        out_specs=pl.BlockSpec((_BLK, D), lambda i: (i, 0)),
    )(partials, partials)


@jax.jit
def kernel(features, structural_indices):
    partials = _seg_sum_sc(features, structural_indices)
    return _combine_partials(partials)
